# baseline (speedup 1.0000x reference)
import functools

import jax
import jax.numpy as jnp
from jax import lax
from jax.experimental import pallas as pl
from jax.experimental.pallas import tpu as pltpu
from jax.experimental.pallas import tpu_sc as plsc

S = 7
B = 2
C = 20
LEN = 5 * B + C
BS = 64
L = 16
NS = 16
NSLAB = S * S
NG = BS // L
NU = NSLAB * NG
NBUF = 4

_f32 = jnp.float32


_INV_S = 1.0 / S


def _unit_losses(pvm, tvm, jv, bv):
    def pcol(c):
        return plsc.load_gather(pvm, [jv + c, bv])

    def tcol(c):
        return plsc.load_gather(tvm, [jv + c, bv])

    tc4 = tcol(4)
    tc9 = tcol(9)
    pc0 = pcol(4)
    pc1 = pcol(9)

    noo_f = jnp.where(tc4 == _f32(0.0), _f32(1.0), _f32(0.0))
    d0 = pc0 - tc4
    d1 = pc1 - tc9
    noo = _f32(0.5) * noo_f * (d0 * d0 + d1 * d1)

    coo = tc4 > _f32(0.0)

    def coo_fn():
        coo_f = jnp.where(coo, _f32(1.0), _f32(0.0))

        tx, ty, tw, th = tcol(0), tcol(1), tcol(2), tcol(3)
        t1x = tx * _f32(_INV_S) - _f32(0.5) * tw
        t2x = tx * _f32(_INV_S) + _f32(0.5) * tw
        t1y = ty * _f32(_INV_S) - _f32(0.5) * th
        t2y = ty * _f32(_INV_S) + _f32(0.5) * th
        a2 = (t2x - t1x) * (t2y - t1y)

        def iou(px, py, pw, ph):
            p1x = px * _f32(_INV_S) - _f32(0.5) * pw
            p2x = px * _f32(_INV_S) + _f32(0.5) * pw
            p1y = py * _f32(_INV_S) - _f32(0.5) * ph
            p2y = py * _f32(_INV_S) + _f32(0.5) * ph
            wx = jnp.maximum(
                jnp.minimum(p2x, t2x) - jnp.maximum(p1x, t1x), _f32(0.0))
            wy = jnp.maximum(
                jnp.minimum(p2y, t2y) - jnp.maximum(p1y, t1y), _f32(0.0))
            inter = wx * wy
            a1 = (p2x - p1x) * (p2y - p1y)
            denom = a1 + a2 - inter
            safe = jnp.where(coo, denom, _f32(1.0))
            return inter / safe

        iou0 = iou(pcol(0), pcol(1), pcol(2), pcol(3))
        iou1 = iou(pcol(5), pcol(6), pcol(7), pcol(8))
        max_iou = jnp.maximum(iou0, iou1)
        resp_c = jnp.where(iou1 > iou0, pc1, pc0)
        dc = resp_c - max_iou
        contain = dc * dc

        cls = jnp.zeros((L,), _f32)
        for c in range(C):
            d = pcol(10 + c) - tcol(10 + c)
            cls = cls + d * d

        return coo_f * (contain + cls)

    cnt = plsc.all_reduce_population_count(coo)
    obj = lax.cond(cnt[0] > 0, coo_fn, lambda: jnp.zeros((L,), _f32))
    return noo + obj


def _sc_body(pred_hbm, tgt_hbm, out_hbm, pvm, tvm, accvm, redvm, shared,
             sem_p, sem_t):
    sid = lax.axis_index("s")
    u0 = 12 * sid + jnp.minimum(sid, 4)
    cnt = jnp.where(sid < 4, 13, 12)
    slab0 = jnp.minimum(u0 // NG, NSLAB - NBUF)

    lastslab = (u0 + cnt - 1) // NG
    need4 = lastslab - slab0 >= 3

    copies = []
    for j in range(3):
        copies.append(pltpu.async_copy(
            pred_hbm.at[slab0 + j], pvm.at[pl.ds(j * LEN, LEN)], sem_p))
        copies.append(pltpu.async_copy(
            tgt_hbm.at[slab0 + j], tvm.at[pl.ds(j * LEN, LEN)], sem_t))

    @pl.when(need4)
    def _():
        pltpu.async_copy(
            pred_hbm.at[slab0 + 3], pvm.at[pl.ds(3 * LEN, LEN)], sem_p)
        pltpu.async_copy(
            tgt_hbm.at[slab0 + 3], tvm.at[pl.ds(3 * LEN, LEN)], sem_t)

    lane = lax.iota(jnp.int32, L)

    def make_body(j):
        jrow = jnp.full((L,), j * LEN, jnp.int32)

        def unit_body(u, acc):
            bv = (u - (slab0 + j) * NG) * L + lane
            return acc + _unit_losses(pvm, tvm, jrow, bv)

        return unit_body

    acc = jnp.zeros((L,), _f32)
    uend = u0 + cnt
    for j in range(3):
        copies[2 * j].wait()
        copies[2 * j + 1].wait()
        lo = jnp.maximum(u0, (slab0 + j) * NG)
        hi = jnp.minimum(uend, (slab0 + j + 1) * NG)
        acc = lax.fori_loop(lo, hi, make_body(j), acc)
    accvm[...] = acc

    @pl.when(need4)
    def _():
        pltpu.make_async_copy(
            pred_hbm.at[slab0 + 3], pvm.at[pl.ds(3 * LEN, LEN)], sem_p).wait()
        pltpu.make_async_copy(
            tgt_hbm.at[slab0 + 3], tvm.at[pl.ds(3 * LEN, LEN)], sem_t).wait()
        lo = jnp.maximum(u0, (slab0 + 3) * NG)
        acc4 = lax.fori_loop(lo, uend, make_body(3), jnp.zeros((L,), _f32))
        accvm[...] = accvm[...] + acc4

    pltpu.sync_copy(accvm, shared.at[pl.ds(sid * L, L)])
    plsc.subcore_barrier()

    @pl.when(sid == 0)
    def _():
        pltpu.sync_copy(shared, redvm)
        t = jnp.zeros((L,), _f32)
        for i in range(NS):
            t = t + redvm[pl.ds(i * L, L)]
        total = jnp.sum(t) * _f32(1.0 / BS)
        accvm[...] = jnp.full((L,), total, _f32)
        pltpu.sync_copy(accvm, out_hbm)


_mesh = plsc.VectorSubcoreMesh(
    core_axis_name="c", subcore_axis_name="s", num_cores=1)

_sc_yolo = functools.partial(
    pl.kernel,
    out_type=jax.ShapeDtypeStruct((L,), _f32),
    mesh=_mesh,
    compiler_params=pltpu.CompilerParams(
        needs_layout_passes=False, use_tc_tiling_on_sc=True),
    scratch_types=[
        pltpu.VMEM((NBUF * LEN, BS), _f32),
        pltpu.VMEM((NBUF * LEN, BS), _f32),
        pltpu.VMEM((L,), _f32),
        pltpu.VMEM((NS * L,), _f32),
        pltpu.VMEM_SHARED((NS * L,), _f32),
        pltpu.SemaphoreType.DMA,
        pltpu.SemaphoreType.DMA,
    ],
)(_sc_body)


def kernel(prediction, target):
    qp = jnp.transpose(prediction, (1, 2, 3, 0)).reshape(NSLAB, LEN, BS)
    qt = jnp.transpose(target, (1, 2, 3, 0)).reshape(NSLAB, LEN, BS)
    out = _sc_yolo(qp, qt)
    return out[0]


# speedup vs baseline: 1.0204x; 1.0204x over previous
import functools

import jax
import jax.numpy as jnp
from jax import lax
from jax.experimental import pallas as pl
from jax.experimental.pallas import tpu as pltpu
from jax.experimental.pallas import tpu_sc as plsc

S = 7
B = 2
C = 20
LEN = 5 * B + C
BS = 64
L = 16
NS = 16
NSLAB = S * S
NG = BS // L
NU = NSLAB * NG
NBUF = 4

_f32 = jnp.float32


_INV_S = 1.0 / S


def _unit_losses(pvm, tvm, jv, bv):
    def pcol(c):
        return plsc.load_gather(pvm, [jv + c, bv])

    def tcol(c):
        return plsc.load_gather(tvm, [jv + c, bv])

    tc4 = tcol(4)
    tc9 = tcol(9)
    pc0 = pcol(4)
    pc1 = pcol(9)

    noo_f = jnp.where(tc4 == _f32(0.0), _f32(1.0), _f32(0.0))
    d0 = pc0 - tc4
    d1 = pc1 - tc9
    noo = _f32(0.5) * noo_f * (d0 * d0 + d1 * d1)

    coo = tc4 > _f32(0.0)

    def coo_fn():
        coo_f = jnp.where(coo, _f32(1.0), _f32(0.0))

        tx, ty, tw, th = tcol(0), tcol(1), tcol(2), tcol(3)
        t1x = tx * _f32(_INV_S) - _f32(0.5) * tw
        t2x = tx * _f32(_INV_S) + _f32(0.5) * tw
        t1y = ty * _f32(_INV_S) - _f32(0.5) * th
        t2y = ty * _f32(_INV_S) + _f32(0.5) * th
        a2 = (t2x - t1x) * (t2y - t1y)

        def iou(px, py, pw, ph):
            p1x = px * _f32(_INV_S) - _f32(0.5) * pw
            p2x = px * _f32(_INV_S) + _f32(0.5) * pw
            p1y = py * _f32(_INV_S) - _f32(0.5) * ph
            p2y = py * _f32(_INV_S) + _f32(0.5) * ph
            wx = jnp.maximum(
                jnp.minimum(p2x, t2x) - jnp.maximum(p1x, t1x), _f32(0.0))
            wy = jnp.maximum(
                jnp.minimum(p2y, t2y) - jnp.maximum(p1y, t1y), _f32(0.0))
            inter = wx * wy
            a1 = (p2x - p1x) * (p2y - p1y)
            denom = a1 + a2 - inter
            safe = jnp.where(coo, denom, _f32(1.0))
            return inter / safe

        iou0 = iou(pcol(0), pcol(1), pcol(2), pcol(3))
        iou1 = iou(pcol(5), pcol(6), pcol(7), pcol(8))
        max_iou = jnp.maximum(iou0, iou1)
        resp_c = jnp.where(iou1 > iou0, pc1, pc0)
        dc = resp_c - max_iou
        contain = dc * dc

        cls = jnp.zeros((L,), _f32)
        for c in range(C):
            d = pcol(10 + c) - tcol(10 + c)
            cls = cls + d * d

        return coo_f * (contain + cls)

    cnt = plsc.all_reduce_population_count(coo)
    obj = lax.cond(cnt[0] > 0, coo_fn, lambda: jnp.zeros((L,), _f32))
    return noo + obj


def _sc_body(pred_hbm, tgt_hbm, out_hbm, pvm, tvm, accvm, redvm, shared,
             sem_p, sem_t):
    sid = lax.axis_index("s")
    u0 = 12 * sid + jnp.minimum(sid, 4)
    cnt = jnp.where(sid < 4, 13, 12)
    slab0 = jnp.minimum(u0 // NG, NSLAB - NBUF)

    copies = []
    for j in range(NBUF):
        copies.append(pltpu.async_copy(
            pred_hbm.at[slab0 + j], pvm.at[pl.ds(j * LEN, LEN)], sem_p))
        copies.append(pltpu.async_copy(
            tgt_hbm.at[slab0 + j], tvm.at[pl.ds(j * LEN, LEN)], sem_t))
    for cp in copies:
        cp.wait()

    lane = lax.iota(jnp.int32, L)

    def unit_body(i, acc):
        u = u0 + i
        slab = u // NG
        jv = jnp.full((L,), (slab - slab0) * LEN, jnp.int32)
        bv = (u - slab * NG) * L + lane
        return acc + _unit_losses(pvm, tvm, jv, bv)

    accvm[...] = lax.fori_loop(0, cnt, unit_body, jnp.zeros((L,), _f32))

    pltpu.sync_copy(accvm, shared.at[pl.ds(sid * L, L)])
    plsc.subcore_barrier()

    @pl.when(sid == 0)
    def _():
        pltpu.sync_copy(shared, redvm)
        t = jnp.zeros((L,), _f32)
        for i in range(NS):
            t = t + redvm[pl.ds(i * L, L)]
        total = jnp.sum(t) * _f32(1.0 / BS)
        accvm[...] = jnp.full((L,), total, _f32)
        pltpu.sync_copy(accvm, out_hbm)


_mesh = plsc.VectorSubcoreMesh(
    core_axis_name="c", subcore_axis_name="s", num_cores=1)

_sc_yolo = functools.partial(
    pl.kernel,
    out_type=jax.ShapeDtypeStruct((L,), _f32),
    mesh=_mesh,
    compiler_params=pltpu.CompilerParams(
        needs_layout_passes=False, use_tc_tiling_on_sc=True),
    scratch_types=[
        pltpu.VMEM((NBUF * LEN, BS), _f32),
        pltpu.VMEM((NBUF * LEN, BS), _f32),
        pltpu.VMEM((L,), _f32),
        pltpu.VMEM((NS * L,), _f32),
        pltpu.VMEM_SHARED((NS * L,), _f32),
        pltpu.SemaphoreType.DMA,
        pltpu.SemaphoreType.DMA,
    ],
)(_sc_body)


def kernel(prediction, target):
    qp = jnp.transpose(prediction, (1, 2, 3, 0)).reshape(NSLAB, LEN, BS)
    qt = jnp.transpose(target, (1, 2, 3, 0)).reshape(NSLAB, LEN, BS)
    out = _sc_yolo(qp, qt)
    return out[0]
